# Initial kernel scaffold; baseline (speedup 1.0000x reference)
import jax
import jax.numpy as jnp
from jax.experimental import pallas as pl


def kernel(pos, h, edge_index, We, be, Wq, bq, Wk, bk, Wv, bv, We0, be0, We1, be1, W1, b1, W2, b2):
    raise NotImplementedError("write your pallas kernel here")


# speedup vs baseline: 12.8076x; 12.8076x over previous
import functools
import math

import jax
import jax.numpy as jnp
from jax import lax
from jax.experimental import pallas as pl
from jax.experimental.pallas import tpu as pltpu
from jax.experimental.pallas import tpu_sc as plsc

NC = 2
NS = 16
NW = NC * NS
CH = 128


def _sc_mesh():
    return plsc.VectorSubcoreMesh(core_axis_name="c", subcore_axis_name="s")


def _sc_gather(qt, kt, src, dst):
    Ep = src.shape[0]
    DQ = qt.shape[1]
    DK = kt.shape[1]
    CHG = 64
    per_w = Ep // NW
    n_ch = per_w // CHG

    @functools.partial(
        pl.kernel,
        mesh=_sc_mesh(),
        out_type=[jax.ShapeDtypeStruct((Ep, DQ), jnp.float32),
                  jax.ShapeDtypeStruct((Ep, DK), jnp.float32)],
        scratch_types=[
            pltpu.VMEM((CHG,), jnp.int32),
            pltpu.VMEM((CHG, DQ), jnp.float32),
            pltpu.VMEM((CHG, DK), jnp.float32),
            pltpu.SemaphoreType.DMA,
        ],
    )
    def kern(qt_hbm, kt_hbm, src_hbm, dst_hbm, gq_hbm, gk_hbm,
             idx_v, bufq_v, bufk_v, sem):
        wid = lax.axis_index("s") * NC + lax.axis_index("c")
        base = wid * per_w

        def body(i, carry):
            off = base + i * CHG
            pltpu.sync_copy(dst_hbm.at[pl.ds(off, CHG)], idx_v)
            pltpu.async_copy(qt_hbm.at[idx_v], bufq_v, sem).wait()
            pltpu.sync_copy(bufq_v, gq_hbm.at[pl.ds(off, CHG), :])
            pltpu.sync_copy(src_hbm.at[pl.ds(off, CHG)], idx_v)
            pltpu.async_copy(kt_hbm.at[idx_v], bufk_v, sem).wait()
            pltpu.sync_copy(bufk_v, gk_hbm.at[pl.ds(off, CHG), :])
            return carry

        lax.fori_loop(0, n_ch, body, 0)

    return kern(qt, kt, src, dst)


def _sc_denom(exa, dst, n_pad):
    Ep = exa.shape[0]
    per_c = Ep // NS
    n_ch_a = per_c // CH
    per_w = Ep // NW
    n_ch_g = per_w // CH
    rows_t = n_pad // NS
    ZR = 128

    @functools.partial(
        pl.kernel,
        mesh=_sc_mesh(),
        out_type=[jax.ShapeDtypeStruct((n_pad, 128), jnp.float32),
                  jax.ShapeDtypeStruct((Ep, 128), jnp.float32)],
        scratch_types=[
            pltpu.VMEM((CH,), jnp.int32),
            pltpu.VMEM((CH, 128), jnp.float32),
            pltpu.VMEM((ZR, 128), jnp.float32),
            pltpu.VMEM_SHARED((n_pad, 128), jnp.float32),
            pltpu.SemaphoreType.DMA,
        ],
    )
    def kern(ex_hbm, dst_hbm, dtab_hbm, gs_hbm, idx_v, vals_v, zbuf, acc, sem):
        cid = lax.axis_index("c")
        sid = lax.axis_index("s")
        z = jnp.zeros((16,), jnp.float32)

        def zrow(i, carry):
            def zcol(j, c2):
                zbuf[i, pl.ds(j * 16, 16)] = z
                return c2
            lax.fori_loop(0, 128 // 16, zcol, 0)
            return carry

        lax.fori_loop(0, ZR, zrow, 0)

        def zchunk(i, carry):
            pltpu.sync_copy(zbuf, acc.at[pl.ds(sid * rows_t + i * ZR, ZR), :])
            return carry

        lax.fori_loop(0, rows_t // ZR, zchunk, 0)
        plsc.subcore_barrier()

        def body(i, carry):
            off = sid * per_c + i * CH
            pltpu.sync_copy(dst_hbm.at[pl.ds(off, CH)], idx_v)
            pltpu.sync_copy(ex_hbm.at[pl.ds(off, CH), :], vals_v)
            pltpu.sync_copy(vals_v, acc.at[idx_v], add=True)
            return carry

        lax.fori_loop(0, n_ch_a, body, 0)
        plsc.subcore_barrier()

        pltpu.sync_copy(acc.at[pl.ds(sid * rows_t, rows_t), :],
                        dtab_hbm.at[pl.ds(sid * rows_t, rows_t), :])
        plsc.subcore_barrier()

        def gbody(i, carry):
            off = (sid * NC + cid) * per_w + i * CH
            pltpu.sync_copy(dst_hbm.at[pl.ds(off, CH)], idx_v)
            pltpu.async_copy(dtab_hbm.at[idx_v], vals_v, sem).wait()
            pltpu.sync_copy(vals_v, gs_hbm.at[pl.ds(off, CH), :])
            return carry

        lax.fori_loop(0, n_ch_g, gbody, 0)

    return kern(exa, dst)


def _sc_scatter_hnode(msg, dst, n_pad):
    Ep = dst.shape[0]
    HD = 128
    per_t = Ep // NS
    n_ch = per_t // CH
    rows_t = n_pad // NS
    ZR = 128

    @functools.partial(
        pl.kernel,
        mesh=_sc_mesh(),
        out_type=[jax.ShapeDtypeStruct((n_pad, 2 * HD), jnp.float32)],
        scratch_types=[
            pltpu.VMEM((CH,), jnp.int32),
            pltpu.VMEM((CH, HD), jnp.float32),
            pltpu.VMEM((ZR, HD), jnp.float32),
            pltpu.VMEM_SHARED((n_pad, HD), jnp.float32),
            pltpu.SemaphoreType.DMA,
        ],
    )
    def kern(m_hbm, dst_hbm, hc_hbm, idx_v, vals_v, zbuf, acc, sem):
        cid = lax.axis_index("c")
        sid = lax.axis_index("s")
        base = sid * per_t
        col = cid * HD

        z = jnp.zeros((16,), jnp.float32)

        def zrow(i, carry):
            def zcol(j, c2):
                zbuf[i, pl.ds(j * 16, 16)] = z
                return c2
            lax.fori_loop(0, HD // 16, zcol, 0)
            return carry

        lax.fori_loop(0, ZR, zrow, 0)

        def zchunk(i, carry):
            pltpu.sync_copy(zbuf, acc.at[pl.ds(sid * rows_t + i * ZR, ZR), :])
            return carry

        lax.fori_loop(0, rows_t // ZR, zchunk, 0)
        plsc.subcore_barrier()

        def body(i, carry):
            off = base + i * CH
            pltpu.sync_copy(dst_hbm.at[pl.ds(off, CH)], idx_v)
            pltpu.sync_copy(m_hbm.at[pl.ds(off, CH), pl.ds(col, HD)], vals_v)
            pltpu.sync_copy(vals_v, acc.at[idx_v], add=True)
            return carry

        lax.fori_loop(0, n_ch, body, 0)
        plsc.subcore_barrier()

        pltpu.sync_copy(acc.at[pl.ds(sid * rows_t, rows_t), :],
                        hc_hbm.at[pl.ds(sid * rows_t, rows_t), pl.ds(col, HD)])

    return kern(msg, dst)


def _tc_node(h, pos128, Wq, bq, Wk, bk, Wv, bv):
    n, d = h.shape
    BN = 400
    grid = n // BN

    def body(h_ref, p_ref, wq, bq_, wk, bk_, wv, bv_, qt_ref, kt_ref):
        x = h_ref[...]
        mu = jnp.mean(x, axis=1, keepdims=True)
        xc = x - mu
        var = jnp.mean(xc * xc, axis=1, keepdims=True)
        hn = xc * lax.rsqrt(var + 1e-6)
        p = p_ref[...]
        q = jnp.dot(hn, wq[...], preferred_element_type=jnp.float32) + bq_[...]
        k = jnp.dot(hn, wk[...], preferred_element_type=jnp.float32) + bk_[...]
        v = jnp.dot(hn, wv[...], preferred_element_type=jnp.float32) + bv_[...]
        qt_ref[...] = jnp.concatenate([q, p], axis=1)
        kt_ref[...] = jnp.concatenate([k, v, p], axis=1)

    w_spec = pl.BlockSpec((d, d), lambda i: (0, 0))
    b_spec = pl.BlockSpec((1, d), lambda i: (0, 0))
    return pl.pallas_call(
        body,
        grid=(grid,),
        in_specs=[pl.BlockSpec((BN, d), lambda i: (i, 0)),
                  pl.BlockSpec((BN, 128), lambda i: (i, 0)),
                  w_spec, b_spec, w_spec, b_spec, w_spec, b_spec],
        out_specs=[pl.BlockSpec((BN, d + 128), lambda i: (i, 0)),
                   pl.BlockSpec((BN, 2 * d + 128), lambda i: (i, 0))],
        out_shape=[jax.ShapeDtypeStruct((n, d + 128), jnp.float32),
                   jax.ShapeDtypeStruct((n, 2 * d + 128), jnp.float32)],
    )(h, pos128, Wq, bq.reshape(1, d), Wk, bk.reshape(1, d),
      Wv, bv.reshape(1, d))


def _tc_edge(gq, gk, We, be, We0, be0, We1, be1, n_edges):
    Ep = gq.shape[0]
    D = 256
    ED = We.shape[0]
    H = 8
    C = D // H
    BE = 1024
    grid = Ep // BE
    inv_sqrt_c = 1.0 / math.sqrt(C)

    def body(gq_ref, gk_ref, we, be_, we0, be0_, we1, be1_, ex_ref, e1_ref):
        pid = pl.program_id(0)
        gq_ = gq_ref[...]
        gk_ = gk_ref[...]
        qd = gq_[:, :D]
        ks = gk_[:, :D]
        dp = gq_[:, D:] - gk_[:, 2 * D:]
        d2 = jnp.sum(dp * dp, axis=1, keepdims=True)
        dist = jnp.sqrt(d2 + 1e-12)
        offs = lax.broadcasted_iota(jnp.int32, (1, ED), 1).astype(jnp.float32)
        dgs = dist - offs
        ef = jnp.exp(-0.5 * dgs * dgs)
        ea = jnp.dot(ef, we[...], preferred_element_type=jnp.float32) + be_[...]
        mu = jnp.mean(ea, axis=1, keepdims=True)
        ec = ea - mu
        var = jnp.mean(ec * ec, axis=1, keepdims=True)
        ean = ec * lax.rsqrt(var + 1e-6)
        e0 = jnp.dot(ean, we0[...], preferred_element_type=jnp.float32) + be0_[...]
        e1_ref[...] = jnp.dot(ean, we1[...], preferred_element_type=jnp.float32) + be1_[...]
        t = qd * ks * e0
        sel = (lax.broadcasted_iota(jnp.int32, (D, H), 0) // C
               == lax.broadcasted_iota(jnp.int32, (D, H), 1)).astype(jnp.float32)
        alpha = jnp.dot(t, sel, preferred_element_type=jnp.float32) * inv_sqrt_c
        ex = jnp.exp(alpha)
        rid = pid * BE + lax.broadcasted_iota(jnp.int32, (BE, 1), 0)
        ex = jnp.where(rid < n_edges, ex, 0.0)
        ex_ref[...] = jnp.concatenate(
            [ex, jnp.zeros((BE, 128 - H), jnp.float32)], axis=1)

    return pl.pallas_call(
        body,
        grid=(grid,),
        in_specs=[pl.BlockSpec((BE, D + 128), lambda i: (i, 0)),
                  pl.BlockSpec((BE, 2 * D + 128), lambda i: (i, 0)),
                  pl.BlockSpec((ED, ED), lambda i: (0, 0)),
                  pl.BlockSpec((1, ED), lambda i: (0, 0)),
                  pl.BlockSpec((ED, D), lambda i: (0, 0)),
                  pl.BlockSpec((1, D), lambda i: (0, 0)),
                  pl.BlockSpec((ED, D), lambda i: (0, 0)),
                  pl.BlockSpec((1, D), lambda i: (0, 0))],
        out_specs=[pl.BlockSpec((BE, 128), lambda i: (i, 0)),
                   pl.BlockSpec((BE, D), lambda i: (i, 0))],
        out_shape=[jax.ShapeDtypeStruct((Ep, 128), jnp.float32),
                   jax.ShapeDtypeStruct((Ep, D), jnp.float32)],
    )(gq, gk, We, be.reshape(1, ED), We0, be0.reshape(1, D),
      We1, be1.reshape(1, D))


def _tc_msg(exa, gsum, gk, e1, n_edges):
    Ep = exa.shape[0]
    D = 256
    H = 8
    C = D // H
    HD = D // 2
    BE = 1024
    grid = Ep // BE

    def body(ex_ref, gs_ref, vs0_ref, vs1_ref, e1_ref, m_ref):
        pid = pl.program_id(0)
        ex = ex_ref[...][:, :H]
        dsum = gs_ref[...][:, :H]
        attn = ex / (dsum + 1e-16)
        selT = (lax.broadcasted_iota(jnp.int32, (H, D), 0)
                == lax.broadcasted_iota(jnp.int32, (H, D), 1) // C
                ).astype(jnp.float32)
        attn_c = jnp.dot(attn, selT, preferred_element_type=jnp.float32)
        vs = jnp.concatenate([vs0_ref[...], vs1_ref[...]], axis=1)
        rid = pid * BE + lax.broadcasted_iota(jnp.int32, (BE, 1), 0)
        live = (rid < n_edges).astype(jnp.float32)
        m_ref[...] = vs * e1_ref[...] * attn_c * live

    s128 = pl.BlockSpec((BE, 128), lambda i: (i, 0))
    return pl.pallas_call(
        body,
        grid=(grid,),
        in_specs=[s128, s128,
                  pl.BlockSpec((BE, HD), lambda i: (i, 2)),
                  pl.BlockSpec((BE, HD), lambda i: (i, 3)),
                  pl.BlockSpec((BE, D), lambda i: (i, 0))],
        out_specs=[pl.BlockSpec((BE, D), lambda i: (i, 0))],
        out_shape=[jax.ShapeDtypeStruct((Ep, D), jnp.float32)],
    )(exa, gsum, gk, gk, e1)


def _tc_ff(h, hnode, W1, b1, W2, b2):
    n, d = h.shape
    d2 = W1.shape[1]
    BN = 400
    grid = n // BN

    def body(h_ref, hn_ref, w1, b1_, w2, b2_, o_ref):
        hn = hn_ref[...]
        f = jnp.dot(hn, w1[...], preferred_element_type=jnp.float32) + b1_[...]
        f = f * (1.0 / (1.0 + jnp.exp(-f)))
        o_ref[...] = h_ref[...] + jnp.dot(
            f, w2[...], preferred_element_type=jnp.float32) + b2_[...]

    r_spec = pl.BlockSpec((BN, d), lambda i: (i, 0))
    return pl.pallas_call(
        body,
        grid=(grid,),
        in_specs=[r_spec, r_spec,
                  pl.BlockSpec((d, d2), lambda i: (0, 0)),
                  pl.BlockSpec((1, d2), lambda i: (0, 0)),
                  pl.BlockSpec((d2, d), lambda i: (0, 0)),
                  pl.BlockSpec((1, d), lambda i: (0, 0))],
        out_specs=r_spec,
        out_shape=jax.ShapeDtypeStruct((n, d), jnp.float32),
    )(h, hnode, W1, b1.reshape(1, d2), W2, b2.reshape(1, d))


def kernel(pos, h, edge_index, We, be, Wq, bq, Wk, bk, Wv, bv,
           We0, be0, We1, be1, W1, b1, W2, b2):
    n = pos.shape[0]
    e = edge_index.shape[1]
    ep = ((e + NW * CH - 1) // (NW * CH)) * (NW * CH)
    n_pad = ((n + NS * 8 - 1) // (NS * 8)) * (NS * 8)

    src = edge_index[0].astype(jnp.int32)
    dst = edge_index[1].astype(jnp.int32)
    if ep != e:
        fill = jnp.arange(ep - e, dtype=jnp.int32) % n
        src = jnp.concatenate([src, fill])
        dst = jnp.concatenate([dst, fill])

    pos128 = jnp.pad(pos, ((0, 0), (0, 128 - pos.shape[1])))

    qt, kt = _tc_node(h, pos128, Wq, bq, Wk, bk, Wv, bv)
    gq, gk = _sc_gather(qt, kt, src, dst)
    exa, e1 = _tc_edge(gq, gk, We, be, We0, be0, We1, be1, e)
    _, gsum = _sc_denom(exa, dst, n_pad)
    msg, = _tc_msg(exa, gsum, gk, e1, e)
    hcat, = _sc_scatter_hnode(msg, dst, n_pad)
    return _tc_ff(h, hcat[:n], W1, b1, W2, b2)
